# baseline (speedup 1.0000x reference)
import jax
import jax.numpy as jnp
from jax import lax
from jax.experimental import pallas as pl
from jax.experimental.pallas import tpu as pltpu
from jax.experimental.pallas import tpu_sc as plsc

_P = 105
_D = 2
_M = _P * _D
_LANES = 16
_LVECS = 7
_MVECS = 14
_TAIL = _P - (_LVECS - 1) * _LANES
_NEG = -1e30

_mesh = plsc.VectorSubcoreMesh(
    core_axis_name="c", subcore_axis_name="s", num_cores=1, num_subcores=2
)


_SIG_OFF = 112


def _body(logits_hbm, mag_hbm, out_hbm, lv, mv):
    sid = lax.axis_index("s")

    @pl.when(sid == 0)
    def _softmax():
        pltpu.sync_copy(logits_hbm, lv.at[pl.ds(0, _P)])
        valid = lax.iota(jnp.int32, _LANES) < _TAIL
        tail = lv[pl.ds(_LANES * (_LVECS - 1), _LANES)]
        lv[pl.ds(_LANES * (_LVECS - 1), _LANES)] = jnp.where(valid, tail, _NEG)

        def _max_body(i, acc):
            return jnp.maximum(acc, lv[pl.ds(_LANES * i, _LANES)])

        vmax = lax.fori_loop(1, _LVECS, _max_body, lv[pl.ds(0, _LANES)])
        m = jnp.broadcast_to(jnp.max(vmax), (_LANES,))

        def _exp_body(i, acc):
            e = jnp.exp(lv[pl.ds(_LANES * i, _LANES)] - m)
            lv[pl.ds(_LANES * i, _LANES)] = e
            return acc + e

        vsum = lax.fori_loop(0, _LVECS, _exp_body, jnp.zeros(_LANES, jnp.float32))
        inv = 1.0 / jnp.broadcast_to(jnp.sum(vsum), (_LANES,))

        def _scale_body(i, _):
            lv[pl.ds(_LANES * i, _LANES)] = lv[pl.ds(_LANES * i, _LANES)] * inv
            return 0

        lax.fori_loop(0, _LVECS, _scale_body, 0)
        pltpu.sync_copy(lv.at[pl.ds(0, _P)], out_hbm.at[pl.ds(0, _P)])

    @pl.when(sid == 1)
    def _sigmoid():
        pltpu.sync_copy(mag_hbm, mv.at[pl.ds(0, _M)])

        def _sig_body(i, _):
            x = mv[pl.ds(_LANES * i, _LANES)]
            mv[pl.ds(_LANES * i, _LANES)] = 1.0 / (1.0 + jnp.exp(-x))
            return 0

        lax.fori_loop(0, _MVECS, _sig_body, 0)
        pltpu.sync_copy(mv.at[pl.ds(0, _M)], out_hbm.at[pl.ds(_SIG_OFF, _M)])


_sc_call = pl.kernel(
    _body,
    out_type=jax.ShapeDtypeStruct((_SIG_OFF + _M,), jnp.float32),
    mesh=_mesh,
    scratch_types=(
        pltpu.VMEM((_LVECS * _LANES,), jnp.float32),
        pltpu.VMEM((_MVECS * _LANES,), jnp.float32),
    ),
    name="optaug_policy_sc",
    compiler_params=pltpu.CompilerParams(
        needs_layout_passes=False, skip_device_barrier=True
    ),
)


def kernel(x, logits, mag_params):
    del x
    out = _sc_call(logits, mag_params.reshape(_M))
    return out[:_P], out[_SIG_OFF : _SIG_OFF + _M].reshape(_P, _D)


# speedup vs baseline: 1.0733x; 1.0733x over previous
import jax
import jax.numpy as jnp
from jax import lax
from jax.experimental import pallas as pl
from jax.experimental.pallas import tpu as pltpu
from jax.experimental.pallas import tpu_sc as plsc

_P = 105
_D = 2
_M = _P * _D
_LANES = 16
_LVECS = 7
_MVECS = 14
_TAIL = _P - (_LVECS - 1) * _LANES
_NEG = -1e30

_mesh = plsc.VectorSubcoreMesh(
    core_axis_name="c", subcore_axis_name="s", num_cores=1, num_subcores=2
)


def _body(logits_hbm, mag_hbm, probs_hbm, sig_hbm, lv, mv):
    sid = lax.axis_index("s")

    @pl.when(sid == 0)
    def _softmax():
        pltpu.sync_copy(logits_hbm, lv.at[pl.ds(0, _P)])
        valid = lax.iota(jnp.int32, _LANES) < _TAIL
        tail = lv[pl.ds(_LANES * (_LVECS - 1), _LANES)]
        lv[pl.ds(_LANES * (_LVECS - 1), _LANES)] = jnp.where(valid, tail, _NEG)

        def _max_body(i, acc):
            return jnp.maximum(acc, lv[pl.ds(_LANES * i, _LANES)])

        vmax = lax.fori_loop(1, _LVECS, _max_body, lv[pl.ds(0, _LANES)])
        m = jnp.broadcast_to(jnp.max(vmax), (_LANES,))

        def _exp_body(i, acc):
            e = jnp.exp(lv[pl.ds(_LANES * i, _LANES)] - m)
            lv[pl.ds(_LANES * i, _LANES)] = e
            return acc + e

        vsum = lax.fori_loop(0, _LVECS, _exp_body, jnp.zeros(_LANES, jnp.float32))
        inv = 1.0 / jnp.broadcast_to(jnp.sum(vsum), (_LANES,))

        def _scale_body(i, _):
            lv[pl.ds(_LANES * i, _LANES)] = lv[pl.ds(_LANES * i, _LANES)] * inv
            return 0

        lax.fori_loop(0, _LVECS, _scale_body, 0)
        pltpu.sync_copy(lv.at[pl.ds(0, _P)], probs_hbm)

    @pl.when(sid == 1)
    def _sigmoid():
        pltpu.sync_copy(mag_hbm, mv.at[pl.ds(0, _M)])

        def _sig_body(i, _):
            x = mv[pl.ds(_LANES * i, _LANES)]
            mv[pl.ds(_LANES * i, _LANES)] = 1.0 / (1.0 + jnp.exp(-x))
            return 0

        lax.fori_loop(0, _MVECS, _sig_body, 0)
        pltpu.sync_copy(mv.at[pl.ds(0, _M)], sig_hbm)


_sc_call = pl.kernel(
    _body,
    out_type=(
        jax.ShapeDtypeStruct((_P,), jnp.float32),
        jax.ShapeDtypeStruct((_M,), jnp.float32),
    ),
    mesh=_mesh,
    scratch_types=(
        pltpu.VMEM((_LVECS * _LANES,), jnp.float32),
        pltpu.VMEM((_MVECS * _LANES,), jnp.float32),
    ),
    name="optaug_policy_sc",
    compiler_params=pltpu.CompilerParams(needs_layout_passes=False),
)


def kernel(x, logits, mag_params):
    del x
    probs, sig = _sc_call(logits, mag_params.reshape(_M))
    return probs, sig.reshape(_P, _D)
